# baseline (speedup 1.0000x reference)
import jax
import jax.numpy as jnp
from jax.experimental import pallas as pl
from jax.experimental.pallas import tpu as pltpu

H, D, HID = 16, 64, 1024
EPS = 1e-6
T = 512


def _fused_kernel(hs_ref, Wq_ref, bq_ref, Wk_ref, bk_ref, Wv_ref, bv_ref,
                  Wo_ref, Mbd_ref, Zseg_ref,
                  out_ref, ktv_ref, zp_ref):
    c = pl.program_id(1)
    f32 = jnp.float32
    hs = hs_ref[0]
    q32 = jnp.dot(hs, Wq_ref[...], preferred_element_type=f32) + bq_ref[...]
    k32 = jnp.dot(hs, Wk_ref[...], preferred_element_type=f32) + bk_ref[...]
    v32 = jnp.dot(hs, Wv_ref[...], preferred_element_type=f32) + bv_ref[...]
    qb = q32.astype(jnp.bfloat16).astype(f32)
    kb = k32.astype(jnp.bfloat16).astype(f32)
    vb = v32.astype(jnp.bfloat16).astype(f32)
    den = jnp.dot(qb, Zseg_ref[...], preferred_element_type=f32)
    num = jnp.dot(qb, Mbd_ref[...], preferred_element_type=f32)
    attn = num / (jnp.abs(den) + EPS)
    attn_b = attn.astype(jnp.bfloat16).astype(f32)
    out_ref[0] = jnp.dot(attn_b, Wo_ref[...], preferred_element_type=f32)
    ktv = jax.lax.dot_general(kb, vb, (((0,), (0,)), ((), ())),
                              preferred_element_type=f32)
    zp = jnp.sum(k32, axis=0, keepdims=True)

    @pl.when(c == 0)
    def _init():
        ktv_ref[0] = ktv
        zp_ref[0] = zp

    @pl.when(c != 0)
    def _accum():
        ktv_ref[0] += ktv
        zp_ref[0] += zp


def kernel(hidden_states, M, z, Wq, bq, Wk, bk, Wv, bv, Wo):
    B, S, _ = hidden_states.shape
    NC = S // T
    f32 = jnp.float32
    eye = jnp.eye(H, dtype=f32)
    Mbd = (eye[:, None, :, None] * M[:, :, None, :]).reshape(HID, HID)
    Zseg = jnp.broadcast_to(eye[:, None, :, None] * z[:, :, None, None],
                            (H, D, H, D)).reshape(HID, HID)
    bq2 = bq.reshape(1, HID)
    bk2 = bk.reshape(1, HID)
    bv2 = bv.reshape(1, HID)

    wspec = pl.BlockSpec((HID, HID), lambda b, c: (0, 0))
    bspec = pl.BlockSpec((1, HID), lambda b, c: (0, 0))
    out, ktv, zp = pl.pallas_call(
        _fused_kernel,
        grid=(B, NC),
        in_specs=[
            pl.BlockSpec((1, T, HID), lambda b, c: (b, c, 0)),
            wspec, bspec, wspec, bspec, wspec, bspec,
            wspec, wspec, wspec,
        ],
        out_specs=[
            pl.BlockSpec((1, T, HID), lambda b, c: (b, c, 0)),
            pl.BlockSpec((1, HID, HID), lambda b, c: (b, 0, 0)),
            pl.BlockSpec((1, 1, HID), lambda b, c: (b, 0, 0)),
        ],
        out_shape=[
            jax.ShapeDtypeStruct((B, S, HID), f32),
            jax.ShapeDtypeStruct((B, HID, HID), f32),
            jax.ShapeDtypeStruct((B, 1, HID), f32),
        ],
        compiler_params=pltpu.CompilerParams(
            dimension_semantics=("parallel", "arbitrary"),
        ),
        name="linear_memory_attention",
    )(hidden_states, Wq, bq2, Wk, bk2, Wv, bv2, Wo, Mbd, Zseg)

    ktv_sum = ktv[0] + ktv[1]
    diag = jnp.diagonal(ktv_sum.reshape(H, D, H, D), axis1=0, axis2=2)
    M_new = M + jnp.moveaxis(diag, -1, 0)
    z_new = z + (zp[0, 0] + zp[1, 0]).reshape(H, D)
    return out, M_new, z_new


# speedup vs baseline: 1.1125x; 1.1125x over previous
import jax
import jax.numpy as jnp
from jax.experimental import pallas as pl
from jax.experimental.pallas import tpu as pltpu

H, D, HID = 16, 64, 1024
EPS = 1e-6
T = 512


def _fused_kernel(hs_ref, Wq_ref, bq_ref, Wk_ref, bk_ref, Wv_ref, bv_ref,
                  Wo_ref, Mbd_ref, Zseg_ref,
                  out_ref, ktv_ref, zp_ref):
    c = pl.program_id(1)
    f32 = jnp.float32
    bf = jnp.bfloat16
    hs = hs_ref[0]
    q32 = jnp.dot(hs, Wq_ref[...], preferred_element_type=f32) + bq_ref[...]
    qb = q32.astype(bf)
    den = jnp.dot(qb.astype(f32), Zseg_ref[...], preferred_element_type=f32)
    num = jnp.dot(qb, Mbd_ref[...], preferred_element_type=f32)
    attn = num / (jnp.abs(den) + EPS)
    attn_b = attn.astype(bf)
    out_ref[0] = jnp.dot(attn_b, Wo_ref[...], preferred_element_type=f32)
    hs_b = hs.astype(bf)
    k32 = jnp.dot(hs_b, Wk_ref[...], preferred_element_type=f32) + bk_ref[...]
    zp = jnp.sum(k32, axis=0, keepdims=True)
    kb = k32.astype(bf)
    v32 = jnp.dot(hs_b, Wv_ref[...], preferred_element_type=f32) + bv_ref[...]
    vb = v32.astype(bf)
    ktv_full = jax.lax.dot_general(kb, vb, (((0,), (0,)), ((), ())),
                                   preferred_element_type=f32)
    ktv = jnp.concatenate(
        [ktv_full[h * D:(h + 1) * D, h * D:(h + 1) * D] for h in range(H)],
        axis=1)

    @pl.when(c == 0)
    def _init():
        ktv_ref[0] = ktv
        zp_ref[0] = zp

    @pl.when(c != 0)
    def _accum():
        ktv_ref[0] += ktv
        zp_ref[0] += zp


def kernel(hidden_states, M, z, Wq, bq, Wk, bk, Wv, bv, Wo):
    B, S, _ = hidden_states.shape
    NC = S // T
    f32 = jnp.float32
    eye = jnp.eye(H, dtype=f32)
    Mbd = (eye[:, None, :, None] * M[:, :, None, :]).reshape(HID, HID)
    Mbd = Mbd.astype(jnp.bfloat16)
    Zseg = jnp.broadcast_to(eye[:, None, :, None] * z[:, :, None, None],
                            (H, D, H, D)).reshape(HID, HID)
    Wo_b = Wo.astype(jnp.bfloat16)
    bq2 = bq.reshape(1, HID)
    bk2 = bk.reshape(1, HID)
    bv2 = bv.reshape(1, HID)
    Wk_b = Wk.astype(jnp.bfloat16)
    Wv_b = Wv.astype(jnp.bfloat16)

    wspec = pl.BlockSpec((HID, HID), lambda b, c: (0, 0))
    bspec = pl.BlockSpec((1, HID), lambda b, c: (0, 0))
    out, ktv, zp = pl.pallas_call(
        _fused_kernel,
        grid=(B, NC),
        in_specs=[
            pl.BlockSpec((1, T, HID), lambda b, c: (b, c, 0)),
            wspec, bspec, wspec, bspec, wspec, bspec,
            wspec, wspec, wspec,
        ],
        out_specs=[
            pl.BlockSpec((1, T, HID), lambda b, c: (b, c, 0)),
            pl.BlockSpec((1, D, HID), lambda b, c: (b, 0, 0)),
            pl.BlockSpec((1, 1, HID), lambda b, c: (b, 0, 0)),
        ],
        out_shape=[
            jax.ShapeDtypeStruct((B, S, HID), f32),
            jax.ShapeDtypeStruct((B, D, HID), f32),
            jax.ShapeDtypeStruct((B, 1, HID), f32),
        ],
        compiler_params=pltpu.CompilerParams(
            dimension_semantics=("parallel", "arbitrary"),
        ),
        name="linear_memory_attention",
    )(hidden_states, Wq, bq2, Wk_b, bk2, Wv_b, bv2, Wo_b, Mbd, Zseg)

    ktv_sum = ktv[0] + ktv[1]
    M_new = M + jnp.moveaxis(ktv_sum.reshape(D, H, D), 1, 0)
    z_new = z + (zp[0, 0] + zp[1, 0]).reshape(H, D)
    return out, M_new, z_new
